# baseline (speedup 1.0000x reference)
import functools

import jax
import jax.numpy as jnp
from jax import lax
from jax.experimental import pallas as pl
from jax.experimental.pallas import tpu as pltpu
from jax.experimental.pallas import tpu_sc as plsc

NC = 2
NS = 16
CHUNK = 48


def _matmul_body(x_ref, w_ref, cj_ref, o_ref):
    o_ref[...] = (
        jnp.dot(x_ref[...], w_ref[...], preferred_element_type=jnp.float32)
        * cj_ref[...]
    )


def _combine_body(p_ref, ci_ref, o_ref):
    o_ref[...] = (p_ref[0] + p_ref[1]) * ci_ref[...]


def _make_sc_kernel(n, n_acc, d, n_chunks):
    del n
    rows_per_tile = n_acc // NS
    mesh = plsc.VectorSubcoreMesh(
        core_axis_name="c", subcore_axis_name="s",
        num_cores=NC, num_subcores=NS)

    @functools.partial(
        pl.kernel,
        mesh=mesh,
        out_type=jax.ShapeDtypeStruct((NC, n_acc, d), jnp.float32),
        scratch_types=[
            pltpu.VMEM((n_chunks * CHUNK,), jnp.int32),
            pltpu.VMEM((n_chunks // 2, 2 * CHUNK), jnp.int32),
            pltpu.VMEM((CHUNK, d), jnp.float32),
            pltpu.VMEM((CHUNK, d), jnp.float32),
            pltpu.VMEM((CHUNK, d), jnp.float32),
            pltpu.VMEM((CHUNK, d), jnp.float32),
            pltpu.VMEM_SHARED((n_acc, d), jnp.float32),
            pltpu.SemaphoreType.DMA,
            pltpu.SemaphoreType.DMA,
            pltpu.SemaphoreType.DMA,
            pltpu.SemaphoreType.DMA,
            pltpu.SemaphoreType.DMA,
            pltpu.SemaphoreType.DMA,
            pltpu.SemaphoreType.DMA,
            pltpu.SemaphoreType.DMA,
        ],
    )
    def sc_kernel(h_hbm, src_hbm, dst_hbm, out_hbm,
                  src_v, dst_v, b0, b1, b2, b3, acc,
                  g0, g1, g2, g3, s0, s1, s2, s3):
        c = lax.axis_index("c")
        s = lax.axis_index("s")
        wid = c * NS + s
        bufs = [b0, b1, b2, b3]
        gsems = [g0, g1, g2, g3]
        ssems = [s0, s1, s2, s3]

        def gather(k, b):
            idx = src_v.at[pl.ds(k * CHUNK, CHUNK)]
            return pltpu.make_async_copy(h_hbm.at[idx], bufs[b % 4],
                                         gsems[b % 4])

        def scatter(row, off, b):
            idx = dst_v.at[row, pl.ds(off * CHUNK, CHUNK)]
            return pltpu.make_async_copy(bufs[b % 4], acc.at[idx],
                                         ssems[b % 4])

        pltpu.sync_copy(src_hbm.at[wid], src_v)
        pltpu.sync_copy(dst_hbm.at[wid], dst_v)

        zv = jnp.zeros((16,), jnp.float32)

        def zero_row(i, carry):
            for jj in range(d // 16):
                b0[i, pl.ds(jj * 16, 16)] = zv
            return carry

        lax.fori_loop(0, CHUNK, zero_row, 0)
        full, tail = divmod(rows_per_tile, CHUNK)
        for k in range(full):
            pltpu.sync_copy(b0, acc.at[pl.ds(s * rows_per_tile + k * CHUNK, CHUNK)])
        if tail:
            pltpu.sync_copy(
                b0.at[pl.ds(0, tail)],
                acc.at[pl.ds(s * rows_per_tile + full * CHUNK, tail)])
        plsc.subcore_barrier()


        def step(k, kr, p):
            kp = k if isinstance(k, int) else p
            gather(k, kp).wait()
            scatter(kr, kp % 2, kp).start(add=True)
            if not isinstance(k, int) or k >= 2:
                km = kp - 2
                scatter(kr - 1, kp % 2, km).wait()
            nk = k + 2
            if isinstance(k, int) and nk >= n_chunks:
                return
            gather(nk, kp + 2).start()

        gather(0, 0).start()
        gather(1, 1).start()
        for k in range(4):
            step(k, k // 2, k)

        @pl.loop(4, n_chunks - 4, step=4)
        def _group(j):
            jr = j // 2
            for p in range(4):
                step(j + p, jr + p // 2, p)

        kt = n_chunks - 4
        ktr = kt // 2
        for p in range(4):
            step(kt + p, ktr + p // 2, p)
        scatter(ktr + 1, 0, kt + 2).wait()
        scatter(ktr + 1, 1, kt + 3).wait()
        plsc.subcore_barrier()

        base = s * rows_per_tile
        pltpu.sync_copy(acc.at[pl.ds(base, rows_per_tile)],
                        out_hbm.at[c, pl.ds(base, rows_per_tile)])

    return sc_kernel


def kernel(x, edge_index, cj, ci, W):
    n, d_in = x.shape
    d_out = W.shape[1]
    e = edge_index.shape[1]

    grid1 = 10
    h = pl.pallas_call(
        _matmul_body,
        grid=(grid1,),
        in_specs=[
            pl.BlockSpec((n // grid1, d_in), lambda i: (i, 0)),
            pl.BlockSpec((d_in, d_out), lambda i: (0, 0)),
            pl.BlockSpec((n // grid1, 1), lambda i: (i, 0)),
        ],
        out_specs=pl.BlockSpec((n // grid1, d_out), lambda i: (i, 0)),
        out_shape=jax.ShapeDtypeStruct((n, d_out), jnp.float32),
    )(x, W, cj)

    n_tiles = NC * NS
    e_tile = e // n_tiles
    n_chunks = -(-e_tile // CHUNK)
    n_chunks = -(-n_chunks // 4) * 4
    e_pad = n_chunks * CHUNK
    pad = e_pad - e_tile
    n_acc = -(-(n + 1) // (NS * 8)) * NS * 8
    src = edge_index[0].reshape(n_tiles, e_tile)
    dst = edge_index[1].reshape(n_tiles, e_tile)
    srcp = jnp.concatenate(
        [src, jnp.zeros((n_tiles, pad), jnp.int32)], axis=1)
    dstp = jnp.concatenate(
        [dst, jnp.full((n_tiles, pad), n, jnp.int32)], axis=1
    ).reshape(n_tiles, n_chunks // 2, 2 * CHUNK)

    partials = _make_sc_kernel(n, n_acc, d_out, n_chunks)(h, srcp, dstp)

    grid3 = 5
    rst = pl.pallas_call(
        _combine_body,
        grid=(grid3,),
        in_specs=[
            pl.BlockSpec((NC, n // grid3, d_out), lambda i: (0, i, 0)),
            pl.BlockSpec((n // grid3, 1), lambda i: (i, 0)),
        ],
        out_specs=pl.BlockSpec((n // grid3, d_out), lambda i: (i, 0)),
        out_shape=jax.ShapeDtypeStruct((n, d_out), jnp.float32),
    )(partials, ci)
    return rst


# speedup vs baseline: 1.5300x; 1.5300x over previous
import functools

import jax
import jax.numpy as jnp
from jax import lax
from jax.experimental import pallas as pl
from jax.experimental.pallas import tpu as pltpu
from jax.experimental.pallas import tpu_sc as plsc

NC = 2
NS = 16
CHUNK = 48


def _matmul_body(x_ref, w_ref, cj_ref, o_ref):
    o_ref[...] = (
        jnp.dot(x_ref[...], w_ref[...], preferred_element_type=jnp.float32)
        * cj_ref[...]
    )


def _combine_body(p_ref, ci_ref, o_ref):
    o_ref[...] = (p_ref[0] + p_ref[1]) * ci_ref[...]


def _make_sc_kernel(n, n_acc, d, e_tile):
    rows_per_tile = n_acc // NS
    n_full, tail = divmod(e_tile, CHUNK)
    assert n_full >= 4 and n_full % 2 == 0 and tail % 8 == 0
    mesh = plsc.VectorSubcoreMesh(
        core_axis_name="c", subcore_axis_name="s",
        num_cores=NC, num_subcores=NS)

    @functools.partial(
        pl.kernel,
        mesh=mesh,
        out_type=jax.ShapeDtypeStruct((NC, n_acc, d), jnp.float32),
        scratch_types=[
            pltpu.VMEM((e_tile,), jnp.int32),
            pltpu.VMEM((e_tile,), jnp.int32),
            pltpu.VMEM((CHUNK, d), jnp.float32),
            pltpu.VMEM((CHUNK, d), jnp.float32),
            pltpu.VMEM_SHARED((n_acc, d), jnp.float32),
            pltpu.SemaphoreType.DMA,
            pltpu.SemaphoreType.DMA,
            pltpu.SemaphoreType.DMA,
            pltpu.SemaphoreType.DMA,
        ],
    )
    def sc_kernel(h_hbm, src_hbm, dst_hbm, out_hbm,
                  src_v, dst_v, b0, b1, acc,
                  g0, g1, s0, s1):
        c = lax.axis_index("c")
        s = lax.axis_index("s")
        wid = c * NS + s
        bufs = [b0, b1]
        gsems = [g0, g1]
        ssems = [s0, s1]

        def gather(j, b, m=CHUNK):
            idx = src_v.at[pl.ds(j * CHUNK, m)]
            dst = bufs[b] if m == CHUNK else bufs[b].at[pl.ds(0, m)]
            return pltpu.make_async_copy(h_hbm.at[idx], dst, gsems[b])

        def scatter(j, b, m=CHUNK):
            idx = dst_v.at[pl.ds(j * CHUNK, m)]
            srcb = bufs[b] if m == CHUNK else bufs[b].at[pl.ds(0, m)]
            return pltpu.make_async_copy(srcb, acc.at[idx], ssems[b])

        base_e = wid * e_tile
        pltpu.sync_copy(src_hbm.at[pl.ds(base_e, e_tile)], src_v)
        pltpu.sync_copy(dst_hbm.at[pl.ds(base_e, e_tile)], dst_v)

        zv = jnp.zeros((16,), jnp.float32)

        def zero_row(i, carry):
            for jj in range(d // 16):
                b0[i, pl.ds(jj * 16, 16)] = zv
            return carry

        lax.fori_loop(0, CHUNK, zero_row, 0)
        zfull, ztail = divmod(rows_per_tile, CHUNK)
        zbase = s * rows_per_tile
        zcopies = []
        for k in range(zfull):
            zcopies.append(pltpu.async_copy(
                b0, acc.at[pl.ds(zbase + k * CHUNK, CHUNK)], ssems[k % 2]))
        if ztail:
            zcopies.append(pltpu.async_copy(
                b0.at[pl.ds(0, ztail)],
                acc.at[pl.ds(zbase + zfull * CHUNK, ztail)], ssems[zfull % 2]))
        for cp in zcopies:
            cp.wait()
        plsc.subcore_barrier()

        gather(0, 0).start()
        gather(0, 0).wait()
        scatter(0, 0).start(add=True)
        gather(1, 1).start()
        gather(1, 1).wait()
        scatter(1, 1).start(add=True)
        scatter(0, 0).wait()
        gather(2, 0).start()

        @pl.loop(2, n_full - 2, step=2)
        def _chunk_pair(j):
            gather(j, 0).wait()
            scatter(j, 0).start(add=True)
            scatter(j - 1, 1).wait()
            gather(j + 1, 1).start()
            gather(j + 1, 1).wait()
            scatter(j + 1, 1).start(add=True)
            scatter(j, 0).wait()
            gather(j + 2, 0).start()

        jl = n_full - 2
        gather(jl, 0).wait()
        scatter(jl, 0).start(add=True)
        scatter(jl - 1, 1).wait()
        gather(jl + 1, 1).start()
        gather(jl + 1, 1).wait()
        scatter(jl + 1, 1).start(add=True)
        scatter(jl, 0).wait()
        if tail:
            gather(n_full, 0, tail).start()
            gather(n_full, 0, tail).wait()
            scatter(n_full, 0, tail).start(add=True)
        scatter(jl + 1, 1).wait()
        if tail:
            scatter(n_full, 0, tail).wait()
        plsc.subcore_barrier()

        pltpu.sync_copy(acc.at[pl.ds(zbase, rows_per_tile)],
                        out_hbm.at[c, pl.ds(zbase, rows_per_tile)])

    return sc_kernel


def kernel(x, edge_index, cj, ci, W):
    n, d_in = x.shape
    d_out = W.shape[1]
    e = edge_index.shape[1]

    grid1 = 10
    h = pl.pallas_call(
        _matmul_body,
        grid=(grid1,),
        in_specs=[
            pl.BlockSpec((n // grid1, d_in), lambda i: (i, 0)),
            pl.BlockSpec((d_in, d_out), lambda i: (0, 0)),
            pl.BlockSpec((n // grid1, 1), lambda i: (i, 0)),
        ],
        out_specs=pl.BlockSpec((n // grid1, d_out), lambda i: (i, 0)),
        out_shape=jax.ShapeDtypeStruct((n, d_out), jnp.float32),
    )(x, W, cj)

    n_tiles = NC * NS
    e_tile = e // n_tiles
    n_acc = -(-n // (NS * 8)) * NS * 8

    partials = _make_sc_kernel(n, n_acc, d_out, e_tile)(
        h, edge_index[0], edge_index[1])

    grid3 = 5
    rst = pl.pallas_call(
        _combine_body,
        grid=(grid3,),
        in_specs=[
            pl.BlockSpec((NC, n // grid3, d_out), lambda i: (0, i, 0)),
            pl.BlockSpec((n // grid3, 1), lambda i: (i, 0)),
        ],
        out_specs=pl.BlockSpec((n // grid3, d_out), lambda i: (i, 0)),
        out_shape=jax.ShapeDtypeStruct((n, d_out), jnp.float32),
    )(partials, ci)
    return rst


# speedup vs baseline: 1.7383x; 1.1362x over previous
import functools

import jax
import jax.numpy as jnp
from jax import lax
from jax.experimental import pallas as pl
from jax.experimental.pallas import tpu as pltpu
from jax.experimental.pallas import tpu_sc as plsc

NC = 2
NS = 16
CHUNK = 64


def _matmul_body(x_ref, w_ref, cj_ref, o_ref):
    o_ref[...] = (
        jnp.dot(x_ref[...], w_ref[...], preferred_element_type=jnp.float32)
        * cj_ref[...]
    )


def _combine_body(p_ref, ci_ref, o_ref):
    o_ref[...] = (p_ref[0] + p_ref[1]) * ci_ref[...]


def _make_sc_kernel(n, n_acc, d, e_tile):
    rows_per_tile = n_acc // NS
    n_full, tail = divmod(e_tile, CHUNK)
    assert n_full >= 4 and n_full % 2 == 0 and tail % 8 == 0
    mesh = plsc.VectorSubcoreMesh(
        core_axis_name="c", subcore_axis_name="s",
        num_cores=NC, num_subcores=NS)

    @functools.partial(
        pl.kernel,
        mesh=mesh,
        out_type=jax.ShapeDtypeStruct((NC, n_acc, d), jnp.float32),
        scratch_types=[
            pltpu.VMEM((e_tile,), jnp.int32),
            pltpu.VMEM((e_tile,), jnp.int32),
            pltpu.VMEM((CHUNK, d), jnp.float32),
            pltpu.VMEM((CHUNK, d), jnp.float32),
            pltpu.VMEM_SHARED((n_acc, d), jnp.float32),
            pltpu.SemaphoreType.DMA,
            pltpu.SemaphoreType.DMA,
            pltpu.SemaphoreType.DMA,
            pltpu.SemaphoreType.DMA,
        ],
    )
    def sc_kernel(h_hbm, src_hbm, dst_hbm, out_hbm,
                  src_v, dst_v, b0, b1, acc,
                  g0, g1, s0, s1):
        c = lax.axis_index("c")
        s = lax.axis_index("s")
        wid = c * NS + s
        bufs = [b0, b1]
        gsems = [g0, g1]
        ssems = [s0, s1]

        def gather(j, b, m=CHUNK):
            idx = src_v.at[pl.ds(j * CHUNK, m)]
            dst = bufs[b] if m == CHUNK else bufs[b].at[pl.ds(0, m)]
            return pltpu.make_async_copy(h_hbm.at[idx], dst, gsems[b])

        def scatter(j, b, m=CHUNK):
            idx = dst_v.at[pl.ds(j * CHUNK, m)]
            srcb = bufs[b] if m == CHUNK else bufs[b].at[pl.ds(0, m)]
            return pltpu.make_async_copy(srcb, acc.at[idx], ssems[b])

        base_e = wid * e_tile
        pltpu.sync_copy(src_hbm.at[pl.ds(base_e, e_tile)], src_v)
        pltpu.sync_copy(dst_hbm.at[pl.ds(base_e, e_tile)], dst_v)

        zv = jnp.zeros((16,), jnp.float32)

        def zero_row(i, carry):
            for jj in range(d // 16):
                b0[i, pl.ds(jj * 16, 16)] = zv
            return carry

        lax.fori_loop(0, CHUNK, zero_row, 0)
        zfull, ztail = divmod(rows_per_tile, CHUNK)
        zbase = s * rows_per_tile
        zcopies = []
        for k in range(zfull):
            zcopies.append(pltpu.async_copy(
                b0, acc.at[pl.ds(zbase + k * CHUNK, CHUNK)], ssems[k % 2]))
        if ztail:
            zcopies.append(pltpu.async_copy(
                b0.at[pl.ds(0, ztail)],
                acc.at[pl.ds(zbase + zfull * CHUNK, ztail)], ssems[zfull % 2]))
        for cp in zcopies:
            cp.wait()
        plsc.subcore_barrier()

        gather(0, 0).start()
        gather(0, 0).wait()
        scatter(0, 0).start(add=True)
        gather(1, 1).start()
        gather(1, 1).wait()
        scatter(1, 1).start(add=True)
        scatter(0, 0).wait()
        gather(2, 0).start()

        @pl.loop(2, n_full - 2, step=2)
        def _chunk_pair(j):
            gather(j, 0).wait()
            scatter(j, 0).start(add=True)
            scatter(j - 1, 1).wait()
            gather(j + 1, 1).start()
            gather(j + 1, 1).wait()
            scatter(j + 1, 1).start(add=True)
            scatter(j, 0).wait()
            gather(j + 2, 0).start()

        jl = n_full - 2
        gather(jl, 0).wait()
        scatter(jl, 0).start(add=True)
        scatter(jl - 1, 1).wait()
        gather(jl + 1, 1).start()
        gather(jl + 1, 1).wait()
        scatter(jl + 1, 1).start(add=True)
        scatter(jl, 0).wait()
        if tail:
            gather(n_full, 0, tail).start()
            gather(n_full, 0, tail).wait()
            scatter(n_full, 0, tail).start(add=True)
        scatter(jl + 1, 1).wait()
        if tail:
            scatter(n_full, 0, tail).wait()
        plsc.subcore_barrier()

        pltpu.sync_copy(acc.at[pl.ds(zbase, rows_per_tile)],
                        out_hbm.at[c, pl.ds(zbase, rows_per_tile)])

    return sc_kernel


def kernel(x, edge_index, cj, ci, W):
    n, d_in = x.shape
    d_out = W.shape[1]
    e = edge_index.shape[1]

    grid1 = 10
    h = pl.pallas_call(
        _matmul_body,
        grid=(grid1,),
        in_specs=[
            pl.BlockSpec((n // grid1, d_in), lambda i: (i, 0)),
            pl.BlockSpec((d_in, d_out), lambda i: (0, 0)),
            pl.BlockSpec((n // grid1, 1), lambda i: (i, 0)),
        ],
        out_specs=pl.BlockSpec((n // grid1, d_out), lambda i: (i, 0)),
        out_shape=jax.ShapeDtypeStruct((n, d_out), jnp.float32),
    )(x, W, cj)

    n_tiles = NC * NS
    e_tile = e // n_tiles
    n_acc = -(-n // (NS * 8)) * NS * 8

    partials = _make_sc_kernel(n, n_acc, d_out, e_tile)(
        h, edge_index[0], edge_index[1])

    grid3 = 5
    rst = pl.pallas_call(
        _combine_body,
        grid=(grid3,),
        in_specs=[
            pl.BlockSpec((NC, n // grid3, d_out), lambda i: (0, i, 0)),
            pl.BlockSpec((n // grid3, 1), lambda i: (i, 0)),
        ],
        out_specs=pl.BlockSpec((n // grid3, d_out), lambda i: (i, 0)),
        out_shape=jax.ShapeDtypeStruct((n, d_out), jnp.float32),
    )(partials, ci)
    return rst


# speedup vs baseline: 2.0102x; 1.1564x over previous
import functools

import jax
import jax.numpy as jnp
from jax import lax
from jax.experimental import pallas as pl
from jax.experimental.pallas import tpu as pltpu
from jax.experimental.pallas import tpu_sc as plsc

NC = 2
NS = 16
CHUNK = 96


def _matmul_body(x_ref, w_ref, cj_ref, o_ref):
    o_ref[...] = (
        jnp.dot(x_ref[...], w_ref[...], preferred_element_type=jnp.float32)
        * cj_ref[...]
    )


def _combine_body(p_ref, ci_ref, o_ref):
    o_ref[...] = (p_ref[0] + p_ref[1]) * ci_ref[...]


def _make_sc_kernel(n, n_acc, d, e_tile):
    rows_per_tile = n_acc // NS
    n_full, tail = divmod(e_tile, CHUNK)
    assert n_full >= 4 and n_full % 2 == 0 and tail % 8 == 0
    mesh = plsc.VectorSubcoreMesh(
        core_axis_name="c", subcore_axis_name="s",
        num_cores=NC, num_subcores=NS)

    @functools.partial(
        pl.kernel,
        mesh=mesh,
        out_type=jax.ShapeDtypeStruct((NC, n_acc, d), jnp.float32),
        scratch_types=[
            pltpu.VMEM((e_tile,), jnp.int32),
            pltpu.VMEM((e_tile,), jnp.int32),
            pltpu.VMEM((CHUNK, d), jnp.float32),
            pltpu.VMEM((CHUNK, d), jnp.float32),
            pltpu.VMEM_SHARED((n_acc, d), jnp.float32),
            pltpu.SemaphoreType.DMA,
            pltpu.SemaphoreType.DMA,
            pltpu.SemaphoreType.DMA,
            pltpu.SemaphoreType.DMA,
        ],
    )
    def sc_kernel(h_hbm, src_hbm, dst_hbm, out_hbm,
                  src_v, dst_v, b0, b1, acc,
                  g0, g1, s0, s1):
        c = lax.axis_index("c")
        s = lax.axis_index("s")
        wid = c * NS + s
        bufs = [b0, b1]
        gsems = [g0, g1]
        ssems = [s0, s1]

        def gather(j, b, m=CHUNK):
            idx = src_v.at[pl.ds(j * CHUNK, m)]
            dst = bufs[b] if m == CHUNK else bufs[b].at[pl.ds(0, m)]
            return pltpu.make_async_copy(h_hbm.at[idx], dst, gsems[b])

        def scatter(j, b, m=CHUNK):
            idx = dst_v.at[pl.ds(j * CHUNK, m)]
            srcb = bufs[b] if m == CHUNK else bufs[b].at[pl.ds(0, m)]
            return pltpu.make_async_copy(srcb, acc.at[idx], ssems[b])

        base_e = wid * e_tile
        pltpu.sync_copy(src_hbm.at[pl.ds(base_e, e_tile)], src_v)
        pltpu.sync_copy(dst_hbm.at[pl.ds(base_e, e_tile)], dst_v)

        zv = jnp.zeros((16,), jnp.float32)

        def zero_row(i, carry):
            for jj in range(d // 16):
                b0[i, pl.ds(jj * 16, 16)] = zv
            return carry

        lax.fori_loop(0, CHUNK, zero_row, 0)
        zfull, ztail = divmod(rows_per_tile, CHUNK)
        zbase = s * rows_per_tile
        zcopies = []
        for k in range(zfull):
            zcopies.append(pltpu.async_copy(
                b0, acc.at[pl.ds(zbase + k * CHUNK, CHUNK)], ssems[k % 2]))
        if ztail:
            zcopies.append(pltpu.async_copy(
                b0.at[pl.ds(0, ztail)],
                acc.at[pl.ds(zbase + zfull * CHUNK, ztail)], ssems[zfull % 2]))
        for cp in zcopies:
            cp.wait()
        plsc.subcore_barrier()

        gather(0, 0).start()
        gather(0, 0).wait()
        scatter(0, 0).start(add=True)
        gather(1, 1).start()
        gather(1, 1).wait()
        scatter(1, 1).start(add=True)
        scatter(0, 0).wait()
        gather(2, 0).start()

        @pl.loop(2, n_full - 2, step=2)
        def _chunk_pair(j):
            gather(j, 0).wait()
            scatter(j, 0).start(add=True)
            scatter(j - 1, 1).wait()
            gather(j + 1, 1).start()
            gather(j + 1, 1).wait()
            scatter(j + 1, 1).start(add=True)
            scatter(j, 0).wait()
            gather(j + 2, 0).start()

        jl = n_full - 2
        gather(jl, 0).wait()
        scatter(jl, 0).start(add=True)
        scatter(jl - 1, 1).wait()
        gather(jl + 1, 1).start()
        gather(jl + 1, 1).wait()
        scatter(jl + 1, 1).start(add=True)
        scatter(jl, 0).wait()
        if tail:
            gather(n_full, 0, tail).start()
            gather(n_full, 0, tail).wait()
            scatter(n_full, 0, tail).start(add=True)
        scatter(jl + 1, 1).wait()
        if tail:
            scatter(n_full, 0, tail).wait()
        plsc.subcore_barrier()

        pltpu.sync_copy(acc.at[pl.ds(zbase, rows_per_tile)],
                        out_hbm.at[c, pl.ds(zbase, rows_per_tile)])

    return sc_kernel


def kernel(x, edge_index, cj, ci, W):
    n, d_in = x.shape
    d_out = W.shape[1]
    e = edge_index.shape[1]

    grid1 = 10
    h = pl.pallas_call(
        _matmul_body,
        grid=(grid1,),
        in_specs=[
            pl.BlockSpec((n // grid1, d_in), lambda i: (i, 0)),
            pl.BlockSpec((d_in, d_out), lambda i: (0, 0)),
            pl.BlockSpec((n // grid1, 1), lambda i: (i, 0)),
        ],
        out_specs=pl.BlockSpec((n // grid1, d_out), lambda i: (i, 0)),
        out_shape=jax.ShapeDtypeStruct((n, d_out), jnp.float32),
    )(x, W, cj)

    n_tiles = NC * NS
    e_tile = e // n_tiles
    n_acc = -(-n // (NS * 8)) * NS * 8

    partials = _make_sc_kernel(n, n_acc, d_out, e_tile)(
        h, edge_index[0], edge_index[1])

    grid3 = 5
    rst = pl.pallas_call(
        _combine_body,
        grid=(grid3,),
        in_specs=[
            pl.BlockSpec((NC, n // grid3, d_out), lambda i: (0, i, 0)),
            pl.BlockSpec((n // grid3, 1), lambda i: (i, 0)),
        ],
        out_specs=pl.BlockSpec((n // grid3, d_out), lambda i: (i, 0)),
        out_shape=jax.ShapeDtypeStruct((n, d_out), jnp.float32),
    )(partials, ci)
    return rst


# speedup vs baseline: 2.0600x; 1.0248x over previous
import functools

import jax
import jax.numpy as jnp
from jax import lax
from jax.experimental import pallas as pl
from jax.experimental.pallas import tpu as pltpu
from jax.experimental.pallas import tpu_sc as plsc

NC = 2
NS = 16
CHUNK = 104


def _matmul_body(x_ref, w_ref, cj_ref, o_ref):
    o_ref[...] = (
        jnp.dot(x_ref[...], w_ref[...], preferred_element_type=jnp.float32)
        * cj_ref[...]
    )


def _combine_body(p_ref, ci_ref, o_ref):
    o_ref[...] = (p_ref[0] + p_ref[1]) * ci_ref[...]


def _make_sc_kernel(n, n_acc, d, e_tile):
    rows_per_tile = n_acc // NS
    n_full, tail = divmod(e_tile, CHUNK)
    assert n_full >= 4 and n_full % 2 == 0 and tail % 8 == 0
    mesh = plsc.VectorSubcoreMesh(
        core_axis_name="c", subcore_axis_name="s",
        num_cores=NC, num_subcores=NS)

    @functools.partial(
        pl.kernel,
        mesh=mesh,
        out_type=jax.ShapeDtypeStruct((NC, n_acc, d), jnp.float32),
        scratch_types=[
            pltpu.VMEM((e_tile,), jnp.int32),
            pltpu.VMEM((e_tile,), jnp.int32),
            pltpu.VMEM((CHUNK, d), jnp.float32),
            pltpu.VMEM((CHUNK, d), jnp.float32),
            pltpu.VMEM_SHARED((n_acc, d), jnp.float32),
            pltpu.SemaphoreType.DMA,
            pltpu.SemaphoreType.DMA,
            pltpu.SemaphoreType.DMA,
            pltpu.SemaphoreType.DMA,
        ],
    )
    def sc_kernel(h_hbm, src_hbm, dst_hbm, out_hbm,
                  src_v, dst_v, b0, b1, acc,
                  g0, g1, s0, s1):
        c = lax.axis_index("c")
        s = lax.axis_index("s")
        wid = c * NS + s
        bufs = [b0, b1]
        gsems = [g0, g1]
        ssems = [s0, s1]

        def gather(j, b, m=CHUNK):
            idx = src_v.at[pl.ds(j * CHUNK, m)]
            dst = bufs[b] if m == CHUNK else bufs[b].at[pl.ds(0, m)]
            return pltpu.make_async_copy(h_hbm.at[idx], dst, gsems[b])

        def scatter(j, b, m=CHUNK):
            idx = dst_v.at[pl.ds(j * CHUNK, m)]
            srcb = bufs[b] if m == CHUNK else bufs[b].at[pl.ds(0, m)]
            return pltpu.make_async_copy(srcb, acc.at[idx], ssems[b])

        base_e = wid * e_tile
        pltpu.sync_copy(src_hbm.at[pl.ds(base_e, e_tile)], src_v)
        pltpu.sync_copy(dst_hbm.at[pl.ds(base_e, e_tile)], dst_v)

        zv = jnp.zeros((16,), jnp.float32)

        def zero_row(i, carry):
            for jj in range(d // 16):
                b0[i, pl.ds(jj * 16, 16)] = zv
            return carry

        lax.fori_loop(0, CHUNK, zero_row, 0)
        zfull, ztail = divmod(rows_per_tile, CHUNK)
        zbase = s * rows_per_tile
        zcopies = []
        for k in range(zfull):
            zcopies.append(pltpu.async_copy(
                b0, acc.at[pl.ds(zbase + k * CHUNK, CHUNK)], ssems[k % 2]))
        if ztail:
            zcopies.append(pltpu.async_copy(
                b0.at[pl.ds(0, ztail)],
                acc.at[pl.ds(zbase + zfull * CHUNK, ztail)], ssems[zfull % 2]))
        for cp in zcopies:
            cp.wait()
        plsc.subcore_barrier()

        gather(0, 0).start()
        gather(0, 0).wait()
        scatter(0, 0).start(add=True)
        gather(1, 1).start()
        gather(1, 1).wait()
        scatter(1, 1).start(add=True)
        scatter(0, 0).wait()
        gather(2, 0).start()

        @pl.loop(2, n_full - 2, step=2)
        def _chunk_pair(j):
            gather(j, 0).wait()
            scatter(j, 0).start(add=True)
            scatter(j - 1, 1).wait()
            gather(j + 1, 1).start()
            gather(j + 1, 1).wait()
            scatter(j + 1, 1).start(add=True)
            scatter(j, 0).wait()
            gather(j + 2, 0).start()

        jl = n_full - 2
        gather(jl, 0).wait()
        scatter(jl, 0).start(add=True)
        scatter(jl - 1, 1).wait()
        gather(jl + 1, 1).start()
        gather(jl + 1, 1).wait()
        scatter(jl + 1, 1).start(add=True)
        scatter(jl, 0).wait()
        if tail:
            gather(n_full, 0, tail).start()
            gather(n_full, 0, tail).wait()
            scatter(n_full, 0, tail).start(add=True)
        scatter(jl + 1, 1).wait()
        if tail:
            scatter(n_full, 0, tail).wait()
        plsc.subcore_barrier()

        pltpu.sync_copy(acc.at[pl.ds(zbase, rows_per_tile)],
                        out_hbm.at[c, pl.ds(zbase, rows_per_tile)])

    return sc_kernel


def kernel(x, edge_index, cj, ci, W):
    n, d_in = x.shape
    d_out = W.shape[1]
    e = edge_index.shape[1]

    grid1 = 10
    h = pl.pallas_call(
        _matmul_body,
        grid=(grid1,),
        in_specs=[
            pl.BlockSpec((n // grid1, d_in), lambda i: (i, 0)),
            pl.BlockSpec((d_in, d_out), lambda i: (0, 0)),
            pl.BlockSpec((n // grid1, 1), lambda i: (i, 0)),
        ],
        out_specs=pl.BlockSpec((n // grid1, d_out), lambda i: (i, 0)),
        out_shape=jax.ShapeDtypeStruct((n, d_out), jnp.float32),
    )(x, W, cj)

    n_tiles = NC * NS
    e_tile = e // n_tiles
    n_acc = -(-n // (NS * 8)) * NS * 8

    partials = _make_sc_kernel(n, n_acc, d_out, e_tile)(
        h, edge_index[0], edge_index[1])

    grid3 = 5
    rst = pl.pallas_call(
        _combine_body,
        grid=(grid3,),
        in_specs=[
            pl.BlockSpec((NC, n // grid3, d_out), lambda i: (0, i, 0)),
            pl.BlockSpec((n // grid3, 1), lambda i: (i, 0)),
        ],
        out_specs=pl.BlockSpec((n // grid3, d_out), lambda i: (i, 0)),
        out_shape=jax.ShapeDtypeStruct((n, d_out), jnp.float32),
    )(partials, ci)
    return rst
